# baseline (speedup 1.0000x reference)
import functools

import jax
import jax.numpy as jnp
from jax.experimental import pallas as pl
from jax.experimental.pallas import tpu as pltpu

IN_CH, N_HID, N_OUT = 32, 64, 64
ALPHA = 0.1
NV, NE, NF = 4096, 12288, 8192

BM = 256
NEB = NE // BM
NVB = NV // BM
N_CELLS = NV + NE + NF


def _leaky(x):
    return jnp.where(x >= 0, x, ALPHA * x)


def _cxn_kernel(gf2e_ref, ge2v_ref, xv_ref, xe_ref, xf_ref,
                wvT_ref, weT_ref, wfT_ref, we2vT_ref, wf2eT_ref,
                bv_ref, be_ref, bf_ref, be2v_ref, bf2e_ref,
                wlinT_ref, blin_ref, out_ref, acc_ref):
    i = pl.program_id(0)

    @pl.when(i == 0)
    def _init():
        pre_f = jnp.dot(xf_ref[...], wfT_ref[...],
                        preferred_element_type=jnp.float32) + bf_ref[...]
        acc_ref[...] = jnp.sum(_leaky(pre_f), axis=0, keepdims=True)

    @pl.when(i < NEB)
    def _edge_phase():
        m = jnp.dot(gf2e_ref[...], xf_ref[...],
                    preferred_element_type=jnp.float32)
        xe_blk = xe_ref[pl.ds(i * BM, BM), :]
        pre = (jnp.dot(xe_blk, weT_ref[...], preferred_element_type=jnp.float32)
               + jnp.dot(m, wf2eT_ref[...], preferred_element_type=jnp.float32)
               + be_ref[...] + bf2e_ref[...])
        acc_ref[...] += jnp.sum(_leaky(pre), axis=0, keepdims=True)

    @pl.when(i >= NEB)
    def _vertex_phase():
        j = i - NEB
        m = jnp.dot(ge2v_ref[...], xe_ref[...],
                    preferred_element_type=jnp.float32)
        xv_blk = xv_ref[pl.ds(j * BM, BM), :]
        pre = (jnp.dot(xv_blk, wvT_ref[...], preferred_element_type=jnp.float32)
               + jnp.dot(m, we2vT_ref[...], preferred_element_type=jnp.float32)
               + bv_ref[...] + be2v_ref[...])
        acc_ref[...] += jnp.sum(_leaky(pre), axis=0, keepdims=True)

    @pl.when(i == NEB + NVB - 1)
    def _readout():
        z = jnp.maximum(acc_ref[...] * (1.0 / N_CELLS), 0.0)
        out_ref[...] = jnp.dot(z, wlinT_ref[...],
                               preferred_element_type=jnp.float32) + blin_ref[...]


@jax.jit
def kernel(xv, xe, xf, Ge2v, Gf2e, Wv, bv, We, be, Wf, bf,
           We2v, be2v, Wf2e, bf2e, Wlin, blin):
    xv2, xe2, xf2 = xv[0], xe[0], xf[0]
    row = lambda b: b.reshape(1, -1)
    const = lambda i: (0, 0)

    grid = NEB + NVB
    out = pl.pallas_call(
        _cxn_kernel,
        grid=(grid,),
        in_specs=[
            pl.BlockSpec((BM, NF), lambda i: (jnp.minimum(i, NEB - 1), 0)),
            pl.BlockSpec((BM, NE), lambda i: (jnp.maximum(i - NEB, 0), 0)),
            pl.BlockSpec((NV, IN_CH), const),
            pl.BlockSpec((NE, IN_CH), const),
            pl.BlockSpec((NF, IN_CH), const),
        ] + [pl.BlockSpec(w.shape, const) for w in
             (Wv.T, We.T, Wf.T, We2v.T, Wf2e.T)]
          + [pl.BlockSpec((1, N_HID), const)] * 5
          + [pl.BlockSpec(Wlin.T.shape, const),
             pl.BlockSpec((1, N_OUT), const)],
        out_specs=pl.BlockSpec((1, N_OUT), const),
        out_shape=jax.ShapeDtypeStruct((1, N_OUT), jnp.float32),
        scratch_shapes=[pltpu.VMEM((1, N_HID), jnp.float32)],
    )(Gf2e, Ge2v, xv2, xe2, xf2,
      Wv.T, We.T, Wf.T, We2v.T, Wf2e.T,
      row(bv), row(be), row(bf), row(be2v), row(bf2e),
      Wlin.T, row(blin))
    return out


# speedup vs baseline: 1.0027x; 1.0027x over previous
import functools

import jax
import jax.numpy as jnp
from jax.experimental import pallas as pl
from jax.experimental.pallas import tpu as pltpu

IN_CH, N_HID, N_OUT = 32, 64, 64
ALPHA = 0.1
NV, NE, NF = 4096, 12288, 8192

BM = 256
NEB = NE // BM
NVB = NV // BM
N_CELLS = NV + NE + NF


def _leaky(x):
    return jnp.where(x >= 0, x, ALPHA * x)


def _cxn_kernel(gf2e_ref, ge2v_ref, xv_ref, xe_ref, xf_ref,
                wvT_ref, weT_ref, wfT_ref, we2vT_ref, wf2eT_ref,
                bv_ref, be_ref, bf_ref, be2v_ref, bf2e_ref,
                wlinT_ref, blin_ref, out_ref, acc_ref):
    i = pl.program_id(0)

    @pl.when(i == 0)
    def _init():
        pre_f = jnp.dot(xf_ref[...], wfT_ref[...],
                        preferred_element_type=jnp.float32) + bf_ref[...]
        acc_ref[...] = jnp.sum(_leaky(pre_f), axis=0, keepdims=True)

    @pl.when(i < NEB)
    def _edge_phase():
        m = jnp.dot(gf2e_ref[...].astype(jnp.bfloat16),
                    xf_ref[...].astype(jnp.bfloat16),
                    preferred_element_type=jnp.float32)
        xe_blk = xe_ref[pl.ds(i * BM, BM), :]
        pre = (jnp.dot(xe_blk, weT_ref[...], preferred_element_type=jnp.float32)
               + jnp.dot(m, wf2eT_ref[...], preferred_element_type=jnp.float32)
               + be_ref[...] + bf2e_ref[...])
        acc_ref[...] += jnp.sum(_leaky(pre), axis=0, keepdims=True)

    @pl.when(i >= NEB)
    def _vertex_phase():
        j = i - NEB
        m = jnp.dot(ge2v_ref[...].astype(jnp.bfloat16),
                    xe_ref[...].astype(jnp.bfloat16),
                    preferred_element_type=jnp.float32)
        xv_blk = xv_ref[pl.ds(j * BM, BM), :]
        pre = (jnp.dot(xv_blk, wvT_ref[...], preferred_element_type=jnp.float32)
               + jnp.dot(m, we2vT_ref[...], preferred_element_type=jnp.float32)
               + bv_ref[...] + be2v_ref[...])
        acc_ref[...] += jnp.sum(_leaky(pre), axis=0, keepdims=True)

    @pl.when(i == NEB + NVB - 1)
    def _readout():
        z = jnp.maximum(acc_ref[...] * (1.0 / N_CELLS), 0.0)
        out_ref[...] = jnp.dot(z, wlinT_ref[...],
                               preferred_element_type=jnp.float32) + blin_ref[...]


@jax.jit
def kernel(xv, xe, xf, Ge2v, Gf2e, Wv, bv, We, be, Wf, bf,
           We2v, be2v, Wf2e, bf2e, Wlin, blin):
    xv2, xe2, xf2 = xv[0], xe[0], xf[0]
    row = lambda b: b.reshape(1, -1)
    const = lambda i: (0, 0)

    grid = NEB + NVB
    out = pl.pallas_call(
        _cxn_kernel,
        grid=(grid,),
        in_specs=[
            pl.BlockSpec((BM, NF), lambda i: (jnp.minimum(i, NEB - 1), 0)),
            pl.BlockSpec((BM, NE), lambda i: (jnp.maximum(i - NEB, 0), 0)),
            pl.BlockSpec((NV, IN_CH), const),
            pl.BlockSpec((NE, IN_CH), const),
            pl.BlockSpec((NF, IN_CH), const),
        ] + [pl.BlockSpec(w.shape, const) for w in
             (Wv.T, We.T, Wf.T, We2v.T, Wf2e.T)]
          + [pl.BlockSpec((1, N_HID), const)] * 5
          + [pl.BlockSpec(Wlin.T.shape, const),
             pl.BlockSpec((1, N_OUT), const)],
        out_specs=pl.BlockSpec((1, N_OUT), const),
        out_shape=jax.ShapeDtypeStruct((1, N_OUT), jnp.float32),
        scratch_shapes=[pltpu.VMEM((1, N_HID), jnp.float32)],
    )(Gf2e, Ge2v, xv2, xe2, xf2,
      Wv.T, We.T, Wf.T, We2v.T, Wf2e.T,
      row(bv), row(be), row(bf), row(be2v), row(bf2e),
      Wlin.T, row(blin))
    return out
